# baseline (speedup 1.0000x reference)
import functools

import jax
import jax.numpy as jnp
from jax.experimental import pallas as pl
from jax.experimental.pallas import tpu as pltpu
from jax.experimental.pallas import tpu_sc as plsc

_BATCH = 16384
_D_STATE = 512
_D_OUT = 512
_EMBED = 16
_PACK = 8
_PACKED_W = _PACK * _EMBED

_TB = 1024
_GW = 128


def _sc_gather_packed(table_packed, idx2d):
    mesh = plsc.VectorSubcoreMesh(core_axis_name="c", subcore_axis_name="s")

    @functools.partial(
        pl.kernel,
        out_type=jax.ShapeDtypeStruct((_BATCH, _PACKED_W), table_packed.dtype),
        mesh=mesh,
    )
    def run(tab_hbm, idx_hbm, out_hbm):
        def body(i_vmem, o_vmem):
            pltpu.sync_copy(tab_hbm.at[i_vmem.at[0]], o_vmem)

        pltpu.emit_pipeline(
            body,
            grid=(_BATCH // _GW,),
            in_specs=[pl.BlockSpec((1, _GW), lambda i: (0, i))],
            out_specs=[pl.BlockSpec((_GW, _PACKED_W), lambda i: (i, 0))],
            core_axis_name=("c", "s"),
            dimension_semantics=(pltpu.PARALLEL,),
        )(idx_hbm, out_hbm)

    return run(table_packed, idx2d)


def _tc_matmul_select_concat(state, W_enc, gathered, idx_col):
    def body(s_ref, w_ref, g_ref, i_ref, o_ref):
        s = s_ref[...].astype(jnp.bfloat16)
        w = w_ref[...].astype(jnp.bfloat16)
        o_ref[:, :_D_OUT] = jnp.dot(s, w, preferred_element_type=jnp.float32)
        sub = i_ref[...] & (_PACK - 1)
        acc = jnp.zeros((s_ref.shape[0], _EMBED), jnp.float32)
        for k in range(_PACK):
            mask = (sub == k).astype(jnp.float32)
            acc = acc + mask * g_ref[:, k * _EMBED:(k + 1) * _EMBED]
        o_ref[:, _D_OUT:] = acc

    return pl.pallas_call(
        body,
        grid=(_BATCH // _TB,),
        in_specs=[
            pl.BlockSpec((_TB, _D_STATE), lambda i: (i, 0)),
            pl.BlockSpec((_D_STATE, _D_OUT), lambda i: (0, 0)),
            pl.BlockSpec((_TB, _PACKED_W), lambda i: (i, 0)),
            pl.BlockSpec((_TB, 1), lambda i: (i, 0)),
        ],
        out_specs=pl.BlockSpec((_TB, _D_OUT + _EMBED), lambda i: (i, 0)),
        out_shape=jax.ShapeDtypeStruct((_BATCH, _D_OUT + _EMBED), jnp.float32),
    )(state, W_enc, gathered, idx_col)


def kernel(state, last_action, rnn_hxs, W_enc, table):
    idx = last_action.astype(jnp.int32)
    table_packed = table.reshape(table.shape[0] // _PACK, _PACKED_W)
    gathered = _sc_gather_packed(table_packed, (idx >> 3).reshape(1, _BATCH))
    out = _tc_matmul_select_concat(state, W_enc, gathered,
                                   idx.reshape(_BATCH, 1))
    return out, rnn_hxs


# speedup vs baseline: 1.5357x; 1.5357x over previous
import functools

import jax
import jax.numpy as jnp
from jax import lax
from jax.experimental import pallas as pl
from jax.experimental.pallas import tpu as pltpu
from jax.experimental.pallas import tpu_sc as plsc

_BATCH = 16384
_D_STATE = 512
_D_OUT = 512
_EMBED = 16

_NW = 32
_BPW = _BATCH // _NW

_TB = 1024


def _sc_gather(table, idx):
    mesh = plsc.VectorSubcoreMesh(core_axis_name="c", subcore_axis_name="s")

    @functools.partial(
        pl.kernel,
        out_type=jax.ShapeDtypeStruct((_BATCH, _EMBED), table.dtype),
        mesh=mesh,
        scratch_types=[
            pltpu.VMEM((_BPW,), jnp.int32),
            pltpu.VMEM((_BPW, _EMBED), jnp.float32),
            pltpu.SemaphoreType.DMA,
            pltpu.SemaphoreType.DMA,
        ],
    )
    def run(tab_hbm, idx_hbm, out_hbm, idx_v, rows_v, sem, osem):
        wid = lax.axis_index("s") * 2 + lax.axis_index("c")
        base = wid * _BPW
        pltpu.async_copy(idx_hbm.at[pl.ds(base, _BPW)], idx_v, sem).wait()

        @pl.loop(0, _BPW, step=16)
        def _(j):
            v = idx_v[pl.ds(j, 16)]
            for k in range(16):
                pltpu.make_async_copy(
                    tab_hbm.at[v[k]], rows_v.at[j + k], sem
                ).start()

        pltpu.make_async_copy(tab_hbm.at[pl.ds(0, _BPW)], rows_v, sem).wait()
        pltpu.async_copy(rows_v, out_hbm.at[pl.ds(base, _BPW)], osem).wait()

    return run(table, idx)


def _tc_matmul_concat(state, W_enc, act):
    def body(s_ref, w_ref, a_ref, o_ref):
        s = s_ref[...].astype(jnp.bfloat16)
        w = w_ref[...].astype(jnp.bfloat16)
        o_ref[:, :_D_OUT] = jnp.dot(s, w, preferred_element_type=jnp.float32)
        o_ref[:, _D_OUT:] = a_ref[...]

    return pl.pallas_call(
        body,
        grid=(_BATCH // _TB,),
        in_specs=[
            pl.BlockSpec((_TB, _D_STATE), lambda i: (i, 0)),
            pl.BlockSpec((_D_STATE, _D_OUT), lambda i: (0, 0)),
            pl.BlockSpec((_TB, _EMBED), lambda i: (i, 0)),
        ],
        out_specs=pl.BlockSpec((_TB, _D_OUT + _EMBED), lambda i: (i, 0)),
        out_shape=jax.ShapeDtypeStruct((_BATCH, _D_OUT + _EMBED), jnp.float32),
    )(state, W_enc, act)


def kernel(state, last_action, rnn_hxs, W_enc, table):
    idx = last_action.astype(jnp.int32)
    act = _sc_gather(table, idx)
    out = _tc_matmul_concat(state, W_enc, act)
    return out, rnn_hxs
